# baseline (speedup 1.0000x reference)
import functools

import jax
import jax.numpy as jnp
from jax import lax
from jax.experimental import pallas as pl
from jax.experimental.pallas import tpu as pltpu
from jax.experimental.pallas import tpu_sc as plsc

BATCH = 4096
HIST = 200
DIM = 32
NC = 2
NS = 16
LANES = 16
NW = NC * NS
B_PER_W = BATCH // NW
CHUNKS = ((0, 104), (104, 96))
INV_HIST = 1.0 / HIST


NBUF = 4
UNROLL = 8


def _body(idx_hbm, table_hbm, out_hbm, idx_v,
          b0, b1, b2, b3, out_v, s0_, s1_, s2_, s3_):
    bufs = (b0, b1, b2, b3)
    sems = (s0_, s1_, s2_, s3_)
    wid = lax.axis_index("s") * NC + lax.axis_index("c")
    base = wid * B_PER_W
    pltpu.sync_copy(idx_hbm.at[pl.ds(pl.multiple_of(base, 8), B_PER_W)], idx_v)

    def start(r, buf, sem):
        for off, n in CHUNKS:
            pltpu.async_copy(
                table_hbm.at[idx_v.at[r, pl.ds(off, n)]],
                buf.at[pl.ds(off, n)],
                sem,
            )

    def drain(buf, sem):
        for off, n in CHUNKS:
            pltpu.make_async_copy(
                table_hbm.at[idx_v.at[0, pl.ds(off, n)]],
                buf.at[pl.ds(off, n)],
                sem,
            ).wait()

    def acc_row(buf):
        def step(j, s):
            sa0, sb0, sa1, sb1 = s
            base_r = j * UNROLL
            for u in range(UNROLL):
                r = base_r + u
                if u % 2 == 0:
                    sa0 = sa0 + buf[r, pl.ds(0, LANES)]
                    sa1 = sa1 + buf[r, pl.ds(LANES, LANES)]
                else:
                    sb0 = sb0 + buf[r, pl.ds(0, LANES)]
                    sb1 = sb1 + buf[r, pl.ds(LANES, LANES)]
            return sa0, sb0, sa1, sb1

        z = jnp.zeros((LANES,), jnp.float32)
        sa0, sb0, sa1, sb1 = lax.fori_loop(0, HIST // UNROLL, step, (z, z, z, z))
        return (sa0 + sb0) * INV_HIST, (sa1 + sb1) * INV_HIST

    for s in range(NBUF):
        start(s, bufs[s], sems[s])

    def outer(i, carry):
        g = i * NBUF
        for s in range(NBUF):
            r = g + s
            drain(bufs[s], sems[s])
            m0, m1 = acc_row(bufs[s])
            out_v[r, pl.ds(0, LANES)] = m0
            out_v[r, pl.ds(LANES, LANES)] = m1
            rp = r + NBUF

            @pl.when(rp < B_PER_W)
            def _():
                start(rp, bufs[s], sems[s])

        return carry

    lax.fori_loop(0, B_PER_W // NBUF, outer, 0)
    pltpu.sync_copy(out_v, out_hbm.at[pl.ds(pl.multiple_of(base, 8), B_PER_W)])


_mesh = plsc.VectorSubcoreMesh(
    core_axis_name="c", subcore_axis_name="s", num_cores=NC, num_subcores=NS
)

_sc_call = functools.partial(
    pl.kernel,
    out_type=jax.ShapeDtypeStruct((BATCH, DIM), jnp.float32),
    mesh=_mesh,
    scratch_types=(
        [pltpu.VMEM((B_PER_W, HIST), jnp.int32)]
        + [pltpu.VMEM((HIST, DIM), jnp.float32) for _ in range(NBUF)]
        + [pltpu.VMEM((B_PER_W, DIM), jnp.float32)]
        + [pltpu.SemaphoreType.DMA for _ in range(NBUF)]
    ),
    compiler_params=pltpu.CompilerParams(use_tc_tiling_on_sc=False),
)(_body)


@jax.jit
def kernel(inputs, table):
    return _sc_call(inputs.astype(jnp.int32), table)
